# baseline (speedup 1.0000x reference)
import functools

import jax
import jax.numpy as jnp
from jax import lax
from jax.experimental import pallas as pl
from jax.experimental.pallas import tpu as pltpu
from jax.experimental.pallas import tpu_sc as plsc

B = 16384
D = 64
NC = 2
NS = 16
L = 16
NW = NC * NS
BPW = B // NW
IDX_CHUNK = 128
NCHUNK = BPW // IDX_CHUNK


def _sc_body(u_idx, p_idx, n_idx, utab, itab, out_pos, out_neg,
             idx_v, rows_u, rows_p, rows_n, acc_pos_v, acc_neg_v, sem):
    wid = lax.axis_index("s") * NC + lax.axis_index("c")
    base = wid * BPW

    for t, src in enumerate((u_idx, p_idx, n_idx)):
        for j in range(NCHUNK):
            pltpu.sync_copy(src.at[pl.ds(base + j * IDX_CHUNK, IDX_CHUNK)],
                            idx_v.at[t * NCHUNK + j])

    copies = []
    for j in range(NCHUNK):
        dst = pl.ds(j * IDX_CHUNK, IDX_CHUNK)
        copies.append(pltpu.async_copy(utab.at[idx_v.at[0 * NCHUNK + j]],
                                       rows_u.at[dst], sem))
        copies.append(pltpu.async_copy(itab.at[idx_v.at[1 * NCHUNK + j]],
                                       rows_p.at[dst], sem))
        copies.append(pltpu.async_copy(itab.at[idx_v.at[2 * NCHUNK + j]],
                                       rows_n.at[dst], sem))
    for c in copies:
        c.wait()

    lanes = lax.iota(jnp.int32, L)

    def group(g, _):
        accp = jnp.zeros((L,), jnp.float32)
        accn = jnp.zeros((L,), jnp.float32)
        for j in range(L):
            e = g * L + j
            sp = jnp.zeros((L,), jnp.float32)
            sn = jnp.zeros((L,), jnp.float32)
            for c in range(D // L):
                sl = pl.ds(c * L, L)
                vu = rows_u[e, sl]
                sp = sp + vu * rows_p[e, sl]
                sn = sn + vu * rows_n[e, sl]
            m = lanes == j
            accp = jnp.where(m, jnp.sum(sp), accp)
            accn = jnp.where(m, jnp.sum(sn), accn)
        acc_pos_v[pl.ds(g * L, L)] = accp
        acc_neg_v[pl.ds(g * L, L)] = accn
        return 0

    lax.fori_loop(0, BPW // L, group, 0)

    pltpu.sync_copy(acc_pos_v, out_pos.at[pl.ds(base, BPW)])
    pltpu.sync_copy(acc_neg_v, out_neg.at[pl.ds(base, BPW)])


@jax.jit
def kernel(user_inputs, pos_item_inputs, neg_item_inputs, user_table, item_table):
    mesh = plsc.VectorSubcoreMesh(core_axis_name="c", subcore_axis_name="s")
    f = pl.kernel(
        _sc_body,
        out_type=(jax.ShapeDtypeStruct((B,), jnp.float32),
                  jax.ShapeDtypeStruct((B,), jnp.float32)),
        mesh=mesh,
        compiler_params=pltpu.CompilerParams(
            needs_layout_passes=False, use_tc_tiling_on_sc=False),
        scratch_types=[
            pltpu.VMEM((3 * NCHUNK, IDX_CHUNK), jnp.int32),
            pltpu.VMEM((BPW, D), jnp.float32),
            pltpu.VMEM((BPW, D), jnp.float32),
            pltpu.VMEM((BPW, D), jnp.float32),
            pltpu.VMEM((BPW,), jnp.float32),
            pltpu.VMEM((BPW,), jnp.float32),
            pltpu.SemaphoreType.DMA,
        ],
    )
    return f(user_inputs, pos_item_inputs, neg_item_inputs, user_table, item_table)


# speedup vs baseline: 1.5724x; 1.5724x over previous
import functools

import jax
import jax.numpy as jnp
from jax import lax
from jax.experimental import pallas as pl
from jax.experimental.pallas import tpu as pltpu
from jax.experimental.pallas import tpu_sc as plsc

B = 16384
D = 64
NC = 2
NS = 16
L = 16
NW = NC * NS
BPW = B // NW
C = 128
NCH = BPW // C


def _sc_body(u_idx, p_idx, n_idx, utab, itab, out_pos, out_neg,
             idx_su, idx_sp, idx_sn, rows_u, rows_p, rows_n,
             acc_pos_v, acc_neg_v, sems):
    wid = lax.axis_index("s") * NC + lax.axis_index("c")
    base = wid * BPW

    pltpu.sync_copy(u_idx.at[pl.ds(base, BPW)], idx_su)
    pltpu.sync_copy(p_idx.at[pl.ds(base, BPW)], idx_sp)
    pltpu.sync_copy(n_idx.at[pl.ds(base, BPW)], idx_sn)

    lanes = lax.iota(jnp.int32, L)

    def issue(c, cb):
        def body(k, _):
            vu = idx_su[pl.ds(c * C + k * L, L)]
            vp = idx_sp[pl.ds(c * C + k * L, L)]
            vn = idx_sn[pl.ds(c * C + k * L, L)]
            for j in range(L):
                i = k * L + j
                pltpu.async_copy(utab.at[vu[j]], rows_u.at[cb, i], sems.at[cb, 0])
                pltpu.async_copy(itab.at[vp[j]], rows_p.at[cb, i], sems.at[cb, 1])
                pltpu.async_copy(itab.at[vn[j]], rows_n.at[cb, i], sems.at[cb, 2])
            return 0
        lax.fori_loop(0, C // L, body, 0)

    def drain(cb):
        pltpu.make_async_copy(utab.at[pl.ds(0, C)], rows_u.at[cb], sems.at[cb, 0]).wait()
        pltpu.make_async_copy(itab.at[pl.ds(0, C)], rows_p.at[cb], sems.at[cb, 1]).wait()
        pltpu.make_async_copy(itab.at[pl.ds(0, C)], rows_n.at[cb], sems.at[cb, 2]).wait()

    def compute(c, cb):
        def group(g, _):
            accp = jnp.zeros((L,), jnp.float32)
            accn = jnp.zeros((L,), jnp.float32)
            for j in range(L):
                e = g * L + j
                sp = jnp.zeros((L,), jnp.float32)
                sn = jnp.zeros((L,), jnp.float32)
                for cc in range(D // L):
                    sl = pl.ds(cc * L, L)
                    vu = rows_u[cb, e, sl]
                    sp = sp + vu * rows_p[cb, e, sl]
                    sn = sn + vu * rows_n[cb, e, sl]
                m = lanes == j
                accp = jnp.where(m, jnp.sum(sp), accp)
                accn = jnp.where(m, jnp.sum(sn), accn)
            acc_pos_v[pl.ds(c * C + g * L, L)] = accp
            acc_neg_v[pl.ds(c * C + g * L, L)] = accn
            return 0
        lax.fori_loop(0, C // L, group, 0)

    issue(0, 0)
    issue(1, 1)
    for c in range(NCH):
        cb = c % 2
        drain(cb)
        compute(c, cb)
        if c + 2 < NCH:
            issue(c + 2, cb)

    pltpu.sync_copy(acc_pos_v, out_pos.at[pl.ds(base, BPW)])
    pltpu.sync_copy(acc_neg_v, out_neg.at[pl.ds(base, BPW)])


@jax.jit
def kernel(user_inputs, pos_item_inputs, neg_item_inputs, user_table, item_table):
    mesh = plsc.VectorSubcoreMesh(core_axis_name="c", subcore_axis_name="s")
    f = pl.kernel(
        _sc_body,
        out_type=(jax.ShapeDtypeStruct((B,), jnp.float32),
                  jax.ShapeDtypeStruct((B,), jnp.float32)),
        mesh=mesh,
        compiler_params=pltpu.CompilerParams(needs_layout_passes=False),
        scratch_types=[
            pltpu.VMEM((BPW,), jnp.int32),
            pltpu.VMEM((BPW,), jnp.int32),
            pltpu.VMEM((BPW,), jnp.int32),
            pltpu.VMEM((2, C, D), jnp.float32),
            pltpu.VMEM((2, C, D), jnp.float32),
            pltpu.VMEM((2, C, D), jnp.float32),
            pltpu.VMEM((BPW,), jnp.float32),
            pltpu.VMEM((BPW,), jnp.float32),
            pltpu.SemaphoreType.DMA((2, 3)),
        ],
    )
    return f(user_inputs, pos_item_inputs, neg_item_inputs, user_table, item_table)
